# baseline (speedup 1.0000x reference)
import functools

import jax
import jax.numpy as jnp
from jax import lax
from jax.experimental import pallas as pl
from jax.experimental.pallas import tpu as pltpu
from jax.experimental.pallas import tpu_sc as plsc

NC = 2
NS = 16
NW = NC * NS
LANES = 16
CHUNK = 20000


@functools.lru_cache(maxsize=None)
def _sc_partial_matmul(nnz: int, ch: int):
    ew = nnz // NW
    assert ew * NW == nnz and ew % ch == 0 and ch % LANES == 0
    nch = ew // ch
    ngrp = ch // LANES
    mesh = plsc.VectorSubcoreMesh(core_axis_name="c", subcore_axis_name="s")

    @functools.partial(
        pl.kernel,
        mesh=mesh,
        compiler_params=pltpu.CompilerParams(needs_layout_passes=False),
        out_type=jax.ShapeDtypeStruct((NW, 4, LANES), jnp.float32),
        scratch_types=[
            pltpu.VMEM((ch,), jnp.float32),
            pltpu.VMEM((ch,), jnp.float32),
            pltpu.VMEM((2 * ch,), jnp.float32),
            pltpu.VMEM((4, LANES), jnp.float32),
        ],
    )
    def k(t_hbm, w_hbm, out_hbm, t0_v, t1_v, w_v, acc_v):
        wid = lax.axis_index("s") * NC + lax.axis_index("c")
        base = wid * ew
        idx_even = lax.iota(jnp.int32, LANES) * 2
        idx_odd = idx_even + 1
        zero = jnp.zeros((LANES,), jnp.float32)

        def chunk_body(c, accs):
            off = base + c * ch
            pltpu.sync_copy(t_hbm.at[pl.ds(off, ch)], t0_v)
            pltpu.sync_copy(t_hbm.at[pl.ds(nnz + off, ch)], t1_v)
            pltpu.sync_copy(w_hbm.at[pl.ds(2 * off, 2 * ch)], w_v)

            def grp(g, accs2):
                b00, b01, b10, b11 = accs2
                t0 = t0_v[pl.ds(g * LANES, LANES)]
                t1 = t1_v[pl.ds(g * LANES, LANES)]
                gbase = g * (2 * LANES)
                w0 = plsc.load_gather(w_v, [idx_even + gbase])
                w1 = plsc.load_gather(w_v, [idx_odd + gbase])
                return (b00 + t0 * w0, b01 + t0 * w1,
                        b10 + t1 * w0, b11 + t1 * w1)

            return lax.fori_loop(0, ngrp, grp, accs)

        a00, a01, a10, a11 = lax.fori_loop(
            0, nch, chunk_body, (zero, zero, zero, zero))
        acc_v[0, :] = a00
        acc_v[1, :] = a01
        acc_v[2, :] = a10
        acc_v[3, :] = a11
        pltpu.sync_copy(acc_v, out_hbm.at[wid])

    return k


def kernel(temp2, weight, bias):
    nnz = temp2.shape[1]
    t_flat = temp2.reshape(-1)
    w_flat = weight.reshape(-1)
    partials = _sc_partial_matmul(nnz, CHUNK)(t_flat, w_flat)
    s = jnp.sum(partials, axis=(0, 2))
    x = s.reshape(2, 2) + bias
    return jax.nn.relu(x)


# speedup vs baseline: 51.4914x; 51.4914x over previous
import functools

import jax
import jax.numpy as jnp
from jax import lax
from jax.experimental import pallas as pl
from jax.experimental.pallas import tpu as pltpu
from jax.experimental.pallas import tpu_sc as plsc

NC = 2
NS = 16
NW = NC * NS
LANES = 16
TILE = 512
CHUNK = 4096


@functools.lru_cache(maxsize=None)
def _sc_partial_matmul(nnz: int, ch: int):
    assert nnz % TILE == 0 and ch % TILE == 0 and ch % LANES == 0
    total_tiles = nnz // TILE
    tiles_small = total_tiles // NW
    n_big = total_tiles - tiles_small * NW
    mesh = plsc.VectorSubcoreMesh(core_axis_name="c", subcore_axis_name="s")

    @functools.partial(
        pl.kernel,
        mesh=mesh,
        compiler_params=pltpu.CompilerParams(needs_layout_passes=False),
        out_type=jax.ShapeDtypeStruct((NW * 4 * LANES,), jnp.float32),
        scratch_types=[
            pltpu.VMEM((2, ch), jnp.float32),
            pltpu.VMEM((2, ch), jnp.float32),
            pltpu.VMEM((4 * LANES,), jnp.float32),
        ],
    )
    def k(t_hbm, w_hbm, out_hbm, t2_v, w2_v, acc_v):
        wid = lax.axis_index("s") * NC + lax.axis_index("c")
        is_big = (wid < n_big).astype(jnp.int32)
        my_tiles = tiles_small + is_big
        start_tile = wid * tiles_small + jnp.minimum(wid, n_big)
        start_edge = start_tile * TILE
        n_edges = my_tiles * TILE
        nfull = n_edges // ch
        ntail = (n_edges - nfull * ch) // TILE

        zero = jnp.zeros((LANES,), jnp.float32)

        def grp(g, accs2):
            b00, b01, b10, b11 = accs2
            t0 = t2_v[0, pl.ds(g * LANES, LANES)]
            t1 = t2_v[1, pl.ds(g * LANES, LANES)]
            w0 = w2_v[0, pl.ds(g * LANES, LANES)]
            w1 = w2_v[1, pl.ds(g * LANES, LANES)]
            return (b00 + t0 * w0, b01 + t0 * w1,
                    b10 + t1 * w0, b11 + t1 * w1)

        def chunk_body(c, accs):
            off = start_edge + c * ch
            pltpu.sync_copy(t_hbm.at[:, pl.ds(off, ch)], t2_v)
            pltpu.sync_copy(w_hbm.at[:, pl.ds(off, ch)], w2_v)
            return lax.fori_loop(0, ch // LANES, grp, accs)

        def tail_body(t, accs):
            off = start_edge + nfull * ch + t * TILE
            pltpu.sync_copy(t_hbm.at[:, pl.ds(off, TILE)],
                            t2_v.at[:, pl.ds(0, TILE)])
            pltpu.sync_copy(w_hbm.at[:, pl.ds(off, TILE)],
                            w2_v.at[:, pl.ds(0, TILE)])
            return lax.fori_loop(0, TILE // LANES, grp, accs)

        accs = lax.fori_loop(0, nfull, chunk_body, (zero, zero, zero, zero))
        a00, a01, a10, a11 = lax.fori_loop(0, ntail, tail_body, accs)
        acc_v[pl.ds(0, LANES)] = a00
        acc_v[pl.ds(LANES, LANES)] = a01
        acc_v[pl.ds(2 * LANES, LANES)] = a10
        acc_v[pl.ds(3 * LANES, LANES)] = a11
        pltpu.sync_copy(acc_v, out_hbm.at[pl.ds(wid * 4 * LANES, 4 * LANES)])

    return k


def kernel(temp2, weight, bias):
    nnz = temp2.shape[1]
    wt = weight.T
    partials = _sc_partial_matmul(nnz, CHUNK)(temp2, wt)
    s = jnp.sum(partials.reshape(NW, 4, LANES), axis=(0, 2))
    x = s.reshape(2, 2) + bias
    return jax.nn.relu(x)


# speedup vs baseline: 104.4754x; 2.0290x over previous
import functools

import jax
import jax.numpy as jnp
from jax import lax
from jax.experimental import pallas as pl
from jax.experimental.pallas import tpu as pltpu
from jax.experimental.pallas import tpu_sc as plsc

NC = 2
NS = 16
NW = NC * NS
LANES = 16
CHUNK = 2560
UNROLL = 4


@functools.lru_cache(maxsize=None)
def _sc_partial_matmul(nnz: int, ch: int):
    assert nnz % ch == 0 and ch % 128 == 0 and ch % (LANES * UNROLL) == 0
    n_chunks = nnz // ch
    mesh = plsc.VectorSubcoreMesh(core_axis_name="c", subcore_axis_name="s")

    @functools.partial(
        pl.kernel,
        mesh=mesh,
        compiler_params=pltpu.CompilerParams(needs_layout_passes=False),
        out_type=jax.ShapeDtypeStruct((NW * 4 * LANES,), jnp.float32),
        scratch_types=[
            pltpu.VMEM((2, ch), jnp.float32),
            pltpu.VMEM((2, ch), jnp.float32),
            pltpu.VMEM((2, ch), jnp.float32),
            pltpu.VMEM((2, ch), jnp.float32),
            pltpu.VMEM((4 * LANES,), jnp.float32),
            pltpu.SemaphoreType.DMA,
            pltpu.SemaphoreType.DMA,
        ],
    )
    def k(t_hbm, w_hbm, out_hbm, t_a, t_b, w_a, w_b, acc_v, sem_a, sem_b):
        wid = lax.axis_index("s") * NC + lax.axis_index("c")
        my_n = (n_chunks - wid + NW - 1) // NW
        zero = jnp.zeros((LANES,), jnp.float32)
        zeros8 = (zero,) * 8

        def start_dma(i, t_buf, w_buf, sem):
            off = (wid + i * NW) * ch
            pltpu.async_copy(t_hbm.at[:, pl.ds(off, ch)], t_buf, sem)
            pltpu.async_copy(w_hbm.at[:, pl.ds(off, ch)], w_buf, sem)

        def wait_dma(t_buf, w_buf, sem):
            pltpu.make_async_copy(t_hbm.at[:, pl.ds(0, ch)], t_buf, sem).wait()
            pltpu.make_async_copy(w_hbm.at[:, pl.ds(0, ch)], w_buf, sem).wait()

        def compute(t_buf, w_buf, accs):
            def step(s, accs2):
                (a00, a01, a10, a11, b00, b01, b10, b11) = accs2
                for u in range(UNROLL):
                    base = (s * UNROLL + u) * LANES
                    t0 = t_buf[0, pl.ds(base, LANES)]
                    t1 = t_buf[1, pl.ds(base, LANES)]
                    w0 = w_buf[0, pl.ds(base, LANES)]
                    w1 = w_buf[1, pl.ds(base, LANES)]
                    if u % 2 == 0:
                        a00 += t0 * w0
                        a01 += t0 * w1
                        a10 += t1 * w0
                        a11 += t1 * w1
                    else:
                        b00 += t0 * w0
                        b01 += t0 * w1
                        b10 += t1 * w0
                        b11 += t1 * w1
                return (a00, a01, a10, a11, b00, b01, b10, b11)

            return lax.fori_loop(0, ch // (LANES * UNROLL), step, accs)

        start_dma(0, t_a, w_a, sem_a)

        def pair_body(p, accs):
            i0 = 2 * p
            i1 = 2 * p + 1

            @pl.when(i1 < my_n)
            def _():
                start_dma(i1, t_b, w_b, sem_b)

            wait_dma(t_a, w_a, sem_a)
            accs = compute(t_a, w_a, accs)

            @pl.when(i1 + 1 < my_n)
            def _():
                start_dma(i1 + 1, t_a, w_a, sem_a)

            def odd(accs2):
                wait_dma(t_b, w_b, sem_b)
                return compute(t_b, w_b, accs2)

            return lax.cond(i1 < my_n, odd, lambda a: a, accs)

        accs = lax.fori_loop(0, (my_n + 1) // 2, pair_body, zeros8)
        (a00, a01, a10, a11, b00, b01, b10, b11) = accs
        acc_v[pl.ds(0, LANES)] = a00 + b00
        acc_v[pl.ds(LANES, LANES)] = a01 + b01
        acc_v[pl.ds(2 * LANES, LANES)] = a10 + b10
        acc_v[pl.ds(3 * LANES, LANES)] = a11 + b11
        pltpu.sync_copy(acc_v, out_hbm.at[pl.ds(wid * 4 * LANES, 4 * LANES)])

    return k


def kernel(temp2, weight, bias):
    nnz = temp2.shape[1]
    wt = weight.T
    partials = _sc_partial_matmul(nnz, CHUNK)(temp2, wt)
    s = jnp.sum(partials.reshape(NW, 4, LANES), axis=(0, 2))
    x = s.reshape(2, 2) + bias
    return jax.nn.relu(x)
